# baseline (speedup 1.0000x reference)
import functools

import jax
import jax.numpy as jnp
from jax import lax
from jax.experimental import pallas as pl
from jax.experimental.pallas import tpu as pltpu
from jax.experimental.pallas import tpu_sc as plsc

CODEBOOK_SIZE = 1024
CODE_DIM = 256
COMMITMENT_WEIGHT = 0.25

TOK_BLK = 512
NBLK = 9

NC, NS = 2, 16
NW = NC * NS
ROWS_PER_W = (NBLK * TOK_BLK) // NW
CHUNK = 72
NCHUNK = ROWS_PER_W // CHUNK


def _dist_argmin_body(z_ref, cbt_ref, idx_ref, loss_ref):
    i = pl.program_id(0)
    z = z_ref[0]
    cbt = cbt_ref[...]
    cb_sq = jnp.sum(cbt * cbt, axis=0, keepdims=True)
    scores = jnp.dot(z, cbt, preferred_element_type=jnp.float32)
    dist = cb_sq - 2.0 * scores
    min_val = jnp.min(dist, axis=1, keepdims=True)
    lane = lax.broadcasted_iota(jnp.int32, dist.shape, 1)
    idx = jnp.min(jnp.where(dist == min_val, lane, jnp.int32(CODEBOOK_SIZE)),
                  axis=1, keepdims=True)
    idx_ref[0] = idx
    z_sq = jnp.sum(z * z, axis=1, keepdims=True)

    @pl.when(i == 0)
    def _init():
        loss_ref[0, 0] = 0.0

    loss_ref[0, 0] += jnp.sum(min_val + z_sq)

    @pl.when(i == NBLK - 1)
    def _scale():
        total = jnp.float32(NBLK * TOK_BLK * CODE_DIM)
        loss_ref[0, 0] = loss_ref[0, 0] * (
            (1.0 + COMMITMENT_WEIGHT) / total)


_dist_argmin = pl.pallas_call(
    _dist_argmin_body,
    grid=(NBLK,),
    in_specs=[
        pl.BlockSpec((1, TOK_BLK, CODE_DIM), lambda i: (i, 0, 0)),
        pl.BlockSpec((CODE_DIM, CODEBOOK_SIZE), lambda i: (0, 0)),
    ],
    out_specs=[
        pl.BlockSpec((1, TOK_BLK, 1), lambda i: (i, 0, 0)),
        pl.BlockSpec(memory_space=pltpu.SMEM),
    ],
    out_shape=[
        jax.ShapeDtypeStruct((NBLK, TOK_BLK, 1), jnp.int32),
        jax.ShapeDtypeStruct((1, 1), jnp.float32),
    ],
)


@functools.cache
def _make_sc_gather():
    mesh = plsc.VectorSubcoreMesh(core_axis_name="c", subcore_axis_name="s")

    @functools.partial(
        pl.kernel,
        mesh=mesh,
        out_type=jax.ShapeDtypeStruct((NBLK * TOK_BLK, CODE_DIM),
                                      jnp.float32),
        scratch_types=[
            pltpu.VMEM((NCHUNK, CHUNK), jnp.int32),
            pltpu.VMEM((ROWS_PER_W, CODE_DIM), jnp.float32),
            pltpu.SemaphoreType.DMA,
        ],
    )
    def _sc_gather(cb_hbm, idx_hbm, out_hbm, idx_v, rows_v, sem):
        wid = lax.axis_index("s") * NC + lax.axis_index("c")
        base = wid * ROWS_PER_W
        for c in range(NCHUNK):
            pltpu.sync_copy(idx_hbm.at[pl.ds(base + c * CHUNK, CHUNK)],
                            idx_v.at[c])
        copies = [
            pltpu.async_copy(cb_hbm.at[idx_v.at[c]],
                             rows_v.at[pl.ds(c * CHUNK, CHUNK)], sem)
            for c in range(NCHUNK)
        ]
        for cp in copies:
            cp.wait()
        pltpu.sync_copy(rows_v, out_hbm.at[pl.ds(base, ROWS_PER_W)])

    return _sc_gather


def kernel(z, codebook):
    B, N, D = z.shape
    z_blocks = z.reshape(NBLK, TOK_BLK, D)
    idx3, loss_acc = _dist_argmin(z_blocks, codebook.T)
    idx_flat = idx3.reshape(-1)
    q = _make_sc_gather()(codebook, idx_flat)
    quantized_st = q.reshape(B, N, D)
    indices = idx3.reshape(B, N)
    loss = loss_acc[0, 0]
    return quantized_st, indices, loss


# speedup vs baseline: 1.0266x; 1.0266x over previous
import functools

import jax
import jax.numpy as jnp
from jax import lax
from jax.experimental import pallas as pl
from jax.experimental.pallas import tpu as pltpu
from jax.experimental.pallas import tpu_sc as plsc

CODEBOOK_SIZE = 1024
CODE_DIM = 256
COMMITMENT_WEIGHT = 0.25

TOK_BLK = 512
NBLK = 9

NC, NS = 2, 16
NW = NC * NS
ROWS_PER_W = (NBLK * TOK_BLK) // NW
CHUNK = 72
NCHUNK = ROWS_PER_W // CHUNK


def _dist_argmin_body(z_ref, cb_ref, idx_ref, loss_ref, cbt_ref, cbsq_ref):
    i = pl.program_id(0)
    z = z_ref[0]

    @pl.when(i == 0)
    def _prep():
        cb = cb_ref[...]
        cbt_ref[...] = cb.T
        cbsq_ref[...] = jnp.sum(cb * cb, axis=1, keepdims=True).T

    scores = jnp.dot(z, cbt_ref[...], preferred_element_type=jnp.float32)
    dist = cbsq_ref[...] - 2.0 * scores
    min_val = jnp.min(dist, axis=1, keepdims=True)
    lane = lax.broadcasted_iota(jnp.int32, dist.shape, 1)
    idx = jnp.min(jnp.where(dist == min_val, lane, jnp.int32(CODEBOOK_SIZE)),
                  axis=1, keepdims=True)
    idx_ref[0] = idx
    z_sq = jnp.sum(z * z, axis=1, keepdims=True)

    @pl.when(i == 0)
    def _init():
        loss_ref[0, 0] = 0.0

    loss_ref[0, 0] += jnp.sum(min_val + z_sq)

    @pl.when(i == NBLK - 1)
    def _scale():
        total = jnp.float32(NBLK * TOK_BLK * CODE_DIM)
        loss_ref[0, 0] = loss_ref[0, 0] * (
            (1.0 + COMMITMENT_WEIGHT) / total)


_dist_argmin = pl.pallas_call(
    _dist_argmin_body,
    grid=(NBLK,),
    in_specs=[
        pl.BlockSpec((1, TOK_BLK, CODE_DIM), lambda i: (i, 0, 0)),
        pl.BlockSpec((CODEBOOK_SIZE, CODE_DIM), lambda i: (0, 0)),
    ],
    out_specs=[
        pl.BlockSpec((1, TOK_BLK, 1), lambda i: (i, 0, 0)),
        pl.BlockSpec(memory_space=pltpu.SMEM),
    ],
    out_shape=[
        jax.ShapeDtypeStruct((NBLK, TOK_BLK, 1), jnp.int32),
        jax.ShapeDtypeStruct((1, 1), jnp.float32),
    ],
    scratch_shapes=[
        pltpu.VMEM((CODE_DIM, CODEBOOK_SIZE), jnp.float32),
        pltpu.VMEM((1, CODEBOOK_SIZE), jnp.float32),
    ],
)


@functools.cache
def _make_sc_gather():
    mesh = plsc.VectorSubcoreMesh(core_axis_name="c", subcore_axis_name="s")

    @functools.partial(
        pl.kernel,
        mesh=mesh,
        out_type=jax.ShapeDtypeStruct((NBLK * TOK_BLK, CODE_DIM),
                                      jnp.float32),
        scratch_types=[
            pltpu.VMEM((ROWS_PER_W,), jnp.int32),
            pltpu.VMEM((ROWS_PER_W, CODE_DIM), jnp.float32),
            [pltpu.SemaphoreType.DMA] * NCHUNK,
            [pltpu.SemaphoreType.DMA] * NCHUNK,
        ],
    )
    def _sc_gather(cb_hbm, idx_hbm, out_hbm, idx_v, rows_v, gsems, osems):
        wid = lax.axis_index("s") * NC + lax.axis_index("c")
        base = wid * ROWS_PER_W
        pltpu.sync_copy(idx_hbm.at[pl.ds(base, ROWS_PER_W)], idx_v)
        gathers = [
            pltpu.async_copy(cb_hbm.at[idx_v.at[pl.ds(c * CHUNK, CHUNK)]],
                             rows_v.at[pl.ds(c * CHUNK, CHUNK)], gsems[c])
            for c in range(NCHUNK)
        ]
        stores = []
        for c in range(NCHUNK):
            gathers[c].wait()
            stores.append(
                pltpu.async_copy(rows_v.at[pl.ds(c * CHUNK, CHUNK)],
                                 out_hbm.at[pl.ds(base + c * CHUNK, CHUNK)],
                                 osems[c]))
        for st in stores:
            st.wait()

    return _sc_gather


def kernel(z, codebook):
    B, N, D = z.shape
    z_blocks = z.reshape(NBLK, TOK_BLK, D)
    idx3, loss_acc = _dist_argmin(z_blocks, codebook)
    idx_flat = idx3.reshape(-1)
    q = _make_sc_gather()(codebook, idx_flat)
    quantized_st = q.reshape(B, N, D)
    indices = idx3.reshape(B, N)
    loss = loss_acc[0, 0]
    return quantized_st, indices, loss


# speedup vs baseline: 1.0487x; 1.0215x over previous
import functools

import jax
import jax.numpy as jnp
from jax import lax
from jax.experimental import pallas as pl
from jax.experimental.pallas import tpu as pltpu
from jax.experimental.pallas import tpu_sc as plsc

CODEBOOK_SIZE = 1024
CODE_DIM = 256
COMMITMENT_WEIGHT = 0.25

TOK_BLK = 512
NBLK = 9

NC, NS = 2, 16
NW = NC * NS
ROWS_PER_W = (NBLK * TOK_BLK) // NW
CHUNK = 72
NCHUNK = ROWS_PER_W // CHUNK


def _dist_argmin_body(z_ref, cb_ref, idx_ref, loss_ref, cbsq_ref):
    i = pl.program_id(0)
    z = z_ref[0]
    cb = cb_ref[...]

    @pl.when(i == 0)
    def _prep():
        cbsq_ref[...] = jnp.sum(cb * cb, axis=1, keepdims=True)

    scores_t = lax.dot_general(
        cb, z, (((1,), (1,)), ((), ())),
        preferred_element_type=jnp.float32)
    dist_t = cbsq_ref[...] - 2.0 * scores_t
    min_val = jnp.min(dist_t, axis=0, keepdims=True)
    row = lax.broadcasted_iota(jnp.int32, dist_t.shape, 0)
    idx = jnp.min(jnp.where(dist_t == min_val, row, jnp.int32(CODEBOOK_SIZE)),
                  axis=0, keepdims=True)
    idx_ref[0] = idx

    @pl.when(i == 0)
    def _init():
        loss_ref[0, 0] = 0.0

    loss_ref[0, 0] += jnp.sum(min_val) + jnp.sum(z * z)

    @pl.when(i == NBLK - 1)
    def _scale():
        total = jnp.float32(NBLK * TOK_BLK * CODE_DIM)
        loss_ref[0, 0] = loss_ref[0, 0] * (
            (1.0 + COMMITMENT_WEIGHT) / total)


_dist_argmin = pl.pallas_call(
    _dist_argmin_body,
    grid=(NBLK,),
    in_specs=[
        pl.BlockSpec((1, TOK_BLK, CODE_DIM), lambda i: (i, 0, 0)),
        pl.BlockSpec((CODEBOOK_SIZE, CODE_DIM), lambda i: (0, 0)),
    ],
    out_specs=[
        pl.BlockSpec((1, 1, TOK_BLK), lambda i: (i, 0, 0)),
        pl.BlockSpec(memory_space=pltpu.SMEM),
    ],
    out_shape=[
        jax.ShapeDtypeStruct((NBLK, 1, TOK_BLK), jnp.int32),
        jax.ShapeDtypeStruct((1, 1), jnp.float32),
    ],
    scratch_shapes=[
        pltpu.VMEM((CODEBOOK_SIZE, 1), jnp.float32),
    ],
)


@functools.cache
def _make_sc_gather():
    mesh = plsc.VectorSubcoreMesh(core_axis_name="c", subcore_axis_name="s")

    @functools.partial(
        pl.kernel,
        mesh=mesh,
        out_type=jax.ShapeDtypeStruct((NBLK * TOK_BLK, CODE_DIM),
                                      jnp.float32),
        scratch_types=[
            pltpu.VMEM((ROWS_PER_W,), jnp.int32),
            pltpu.VMEM((ROWS_PER_W, CODE_DIM), jnp.float32),
            [pltpu.SemaphoreType.DMA] * NCHUNK,
            [pltpu.SemaphoreType.DMA] * NCHUNK,
        ],
    )
    def _sc_gather(cb_hbm, idx_hbm, out_hbm, idx_v, rows_v, gsems, osems):
        wid = lax.axis_index("s") * NC + lax.axis_index("c")
        base = wid * ROWS_PER_W
        pltpu.sync_copy(idx_hbm.at[pl.ds(base, ROWS_PER_W)], idx_v)
        gathers = [
            pltpu.async_copy(cb_hbm.at[idx_v.at[pl.ds(c * CHUNK, CHUNK)]],
                             rows_v.at[pl.ds(c * CHUNK, CHUNK)], gsems[c])
            for c in range(NCHUNK)
        ]
        stores = []
        for c in range(NCHUNK):
            gathers[c].wait()
            stores.append(
                pltpu.async_copy(rows_v.at[pl.ds(c * CHUNK, CHUNK)],
                                 out_hbm.at[pl.ds(base + c * CHUNK, CHUNK)],
                                 osems[c]))
        for st in stores:
            st.wait()

    return _sc_gather


def kernel(z, codebook):
    B, N, D = z.shape
    z_blocks = z.reshape(NBLK, TOK_BLK, D)
    idx3, loss_acc = _dist_argmin(z_blocks, codebook)
    idx_flat = idx3.reshape(-1)
    q = _make_sc_gather()(codebook, idx_flat)
    quantized_st = q.reshape(B, N, D)
    indices = idx3.reshape(B, N)
    loss = loss_acc[0, 0]
    return quantized_st, indices, loss
